# baseline (speedup 1.0000x reference)
import functools

import jax
import jax.numpy as jnp
from jax import lax
from jax.experimental import pallas as pl
from jax.experimental.pallas import tpu as pltpu
from jax.experimental.pallas import tpu_sc as plsc

NUM_LAYERS = 128
SIZE = 4096
SOFT = 204
INV_DENOM = 1.0 / (2.0 * SOFT)
OFFSET = 3.0
EPS = 1e-5
L = 16
NC, NS = 1, 16
NW = NC * NS
ROWS_PER_W = NUM_LAYERS // NW
NVEC = SIZE // L
WINDOW = 40


def _vrsqrt(v):
    i = plsc.bitcast(v, jnp.int32)
    i = jnp.int32(0x5F3759DF) - lax.shift_right_arithmetic(i, 1)
    y = plsc.bitcast(i, jnp.float32)
    for _ in range(4):
        y = y * (1.5 - 0.5 * v * y * y)
    return y


def _body(p_hbm, w_hbm, b_hbm, masks_hbm, outs_hbm, p_v, w_v, b_v, o_v,
          row_v, sem):
    wid = lax.axis_index("s") * NC + lax.axis_index("c")

    cp_p = pltpu.async_copy(p_hbm.at[0], p_v, sem)
    cp_w = pltpu.async_copy(w_hbm, w_v, sem)
    cp_b = pltpu.async_copy(b_hbm, b_v, sem)
    cp_p.wait()

    acc = jnp.zeros((L,), jnp.float32)
    acc2 = jnp.zeros((L,), jnp.float32)
    xs = []
    for i in range(NUM_LAYERS // L):
        x = p_v[pl.ds(i * L, L)]
        xs.append(x)
        acc = acc + x
        acc2 = acc2 + x * x
    s = jnp.sum(acc)
    s2 = jnp.sum(acc2)
    mu = s * (1.0 / NUM_LAYERS)
    var = s2 * (1.0 / NUM_LAYERS) - mu * mu
    rstd = _vrsqrt(jnp.full((L,), var + EPS, jnp.float32))
    cp_w.wait()
    cp_b.wait()
    for i in range(NUM_LAYERS // L):
        xhat = (xs[i] - mu) * rstd
        y = xhat * w_v[pl.ds(i * L, L)] + b_v[pl.ds(i * L, L)] + OFFSET
        o_v[pl.ds(i * L, L)] = 1.0 / (1.0 + jnp.exp(-y))

    out_cp = None
    if True:
        @pl.when(wid == 0)
        def _():
            pltpu.async_copy(o_v, outs_hbm, sem)

    lane_scaled = (lax.convert_element_type(lax.iota(jnp.int32, L),
                                            jnp.float32) * INV_DENOM)
    ones_v = jnp.full((L,), 1.0, jnp.float32)
    zeros_v = jnp.zeros((L,), jnp.float32)
    copies = []
    for r in range(ROWS_PER_W):
        row = wid * ROWS_PER_W + r
        k_vec = plsc.load_gather(
            o_v, [jnp.full((L,), row, jnp.int32)]) * float(SIZE)
        a_vec = k_vec * INV_DENOM + 0.5 - lane_scaled
        j1_vec = lax.convert_element_type(
            jnp.maximum(k_vec - (SOFT + L - 1.0), 0.0) * (1.0 / L), jnp.int32)
        j1 = jnp.max(j1_vec)
        jsta = jnp.minimum(j1 & ~7, NVEC - WINDOW)
        jsta_f = lax.convert_element_type(jsta, jnp.float32)

        @plsc.parallel_loop(jnp.int32(0), jsta, 8)
        def _fill1(i, r=r):
            base = i * L
            for v in range(8):
                row_v[r, pl.ds(base + v * L, L)] = ones_v

        @plsc.parallel_loop(jsta + WINDOW, jnp.int32(NVEC), 8)
        def _fill0(i, r=r):
            base = i * L
            for v in range(8):
                row_v[r, pl.ds(base + v * L, L)] = zeros_v

        aw_vec = a_vec - jsta_f * (L * INV_DENOM)
        wbase = jsta * L

        @plsc.parallel_loop(jnp.int32(0), jnp.int32(WINDOW), 1, unroll=8)
        def _window(j, r=r, aw_vec=aw_vec, wbase=wbase):
            jf = lax.convert_element_type(j, jnp.float32)
            val = jnp.clip(aw_vec - jf * (L * INV_DENOM), 0.0, 1.0)
            row_v[r, pl.ds(wbase + j * L, L)] = val

        copies.append(pltpu.async_copy(row_v.at[r], masks_hbm.at[row], sem))

    for c in copies:
        c.wait()

    @pl.when(wid == 0)
    def _():
        pltpu.make_async_copy(o_v, outs_hbm, sem).wait()


_sk = functools.partial(
    pl.kernel,
    out_type=(
        jax.ShapeDtypeStruct((NUM_LAYERS, SIZE), jnp.float32),
        jax.ShapeDtypeStruct((NUM_LAYERS,), jnp.float32),
    ),
    mesh=plsc.VectorSubcoreMesh(core_axis_name="c", subcore_axis_name="s",
                                num_cores=NC, num_subcores=NS),
    compiler_params=pltpu.CompilerParams(
        needs_layout_passes=False,
        skip_device_barrier=True,
        disable_bounds_checks=True,
        disable_semaphore_checks=True,
    ),
    scratch_types=[
        pltpu.VMEM((NUM_LAYERS,), jnp.float32),
        pltpu.VMEM((NUM_LAYERS,), jnp.float32),
        pltpu.VMEM((NUM_LAYERS,), jnp.float32),
        pltpu.VMEM((NUM_LAYERS,), jnp.float32),
        pltpu.VMEM((ROWS_PER_W, SIZE), jnp.float32),
        pltpu.SemaphoreType.DMA,
    ],
)(_body)


@jax.jit
def kernel(params, ln_weight, ln_bias):
    masks, outputs = _sk(params, ln_weight, ln_bias)
    return masks, outputs


# speedup vs baseline: 1.0331x; 1.0331x over previous
import functools

import jax
import jax.numpy as jnp
from jax import lax
from jax.experimental import pallas as pl
from jax.experimental.pallas import tpu as pltpu
from jax.experimental.pallas import tpu_sc as plsc

NUM_LAYERS = 128
SIZE = 4096
SOFT = 204
INV_DENOM = 1.0 / (2.0 * SOFT)
OFFSET = 3.0
EPS = 1e-5
L = 16
NC, NS = 2, 16
NW = NC * NS
ROWS_PER_W = NUM_LAYERS // NW
NVEC = SIZE // L
WINDOW = 40


def _vrsqrt(v):
    i = plsc.bitcast(v, jnp.int32)
    i = jnp.int32(0x5F3759DF) - lax.shift_right_arithmetic(i, 1)
    y = plsc.bitcast(i, jnp.float32)
    for _ in range(4):
        y = y * (1.5 - 0.5 * v * y * y)
    return y


def _body(p_hbm, w_hbm, b_hbm, masks_hbm, outs_hbm, p_v, w_v, b_v, o_v,
          row_v, sem):
    wid = lax.axis_index("s") * NC + lax.axis_index("c")

    cp_p = pltpu.async_copy(p_hbm.at[0], p_v, sem)
    cp_w = pltpu.async_copy(w_hbm, w_v, sem)
    cp_b = pltpu.async_copy(b_hbm, b_v, sem)
    cp_p.wait()

    acc = jnp.zeros((L,), jnp.float32)
    acc2 = jnp.zeros((L,), jnp.float32)
    xs = []
    for i in range(NUM_LAYERS // L):
        x = p_v[pl.ds(i * L, L)]
        xs.append(x)
        acc = acc + x
        acc2 = acc2 + x * x
    s = jnp.sum(acc)
    s2 = jnp.sum(acc2)
    mu = s * (1.0 / NUM_LAYERS)
    var = s2 * (1.0 / NUM_LAYERS) - mu * mu
    rstd = _vrsqrt(jnp.full((L,), var + EPS, jnp.float32))
    cp_w.wait()
    cp_b.wait()
    for i in range(NUM_LAYERS // L):
        xhat = (xs[i] - mu) * rstd
        y = xhat * w_v[pl.ds(i * L, L)] + b_v[pl.ds(i * L, L)] + OFFSET
        o_v[pl.ds(i * L, L)] = 1.0 / (1.0 + jnp.exp(-y))

    out_cp = None
    if True:
        @pl.when(wid == 0)
        def _():
            pltpu.async_copy(o_v, outs_hbm, sem)

    lane_scaled = (lax.convert_element_type(lax.iota(jnp.int32, L),
                                            jnp.float32) * INV_DENOM)
    ones_v = jnp.full((L,), 1.0, jnp.float32)
    zeros_v = jnp.zeros((L,), jnp.float32)
    copies = []
    for r in range(ROWS_PER_W):
        row = wid * ROWS_PER_W + r
        k_vec = plsc.load_gather(
            o_v, [jnp.full((L,), row, jnp.int32)]) * float(SIZE)
        a_vec = k_vec * INV_DENOM + 0.5 - lane_scaled
        j1_vec = lax.convert_element_type(
            jnp.maximum(k_vec - (SOFT + L - 1.0), 0.0) * (1.0 / L), jnp.int32)
        j1 = jnp.max(j1_vec)
        jsta = jnp.minimum(j1 & ~7, NVEC - WINDOW)
        jsta_f = lax.convert_element_type(jsta, jnp.float32)

        @plsc.parallel_loop(jnp.int32(0), jsta, 8)
        def _fill1(i, r=r):
            base = i * L
            for v in range(8):
                row_v[r, pl.ds(base + v * L, L)] = ones_v

        @plsc.parallel_loop(jsta + WINDOW, jnp.int32(NVEC), 8)
        def _fill0(i, r=r):
            base = i * L
            for v in range(8):
                row_v[r, pl.ds(base + v * L, L)] = zeros_v

        aw_vec = a_vec - jsta_f * (L * INV_DENOM)
        wbase = jsta * L

        @plsc.parallel_loop(jnp.int32(0), jnp.int32(WINDOW), 1, unroll=8)
        def _window(j, r=r, aw_vec=aw_vec, wbase=wbase):
            jf = lax.convert_element_type(j, jnp.float32)
            val = jnp.clip(aw_vec - jf * (L * INV_DENOM), 0.0, 1.0)
            row_v[r, pl.ds(wbase + j * L, L)] = val

        copies.append(pltpu.async_copy(row_v.at[r], masks_hbm.at[row], sem))

    for c in copies:
        c.wait()

    @pl.when(wid == 0)
    def _():
        pltpu.make_async_copy(o_v, outs_hbm, sem).wait()


_sk = functools.partial(
    pl.kernel,
    out_type=(
        jax.ShapeDtypeStruct((NUM_LAYERS, SIZE), jnp.float32),
        jax.ShapeDtypeStruct((NUM_LAYERS,), jnp.float32),
    ),
    mesh=plsc.VectorSubcoreMesh(core_axis_name="c", subcore_axis_name="s",
                                num_cores=NC, num_subcores=NS),
    compiler_params=pltpu.CompilerParams(
        needs_layout_passes=False,
        skip_device_barrier=True,
        disable_bounds_checks=True,
        disable_semaphore_checks=True,
    ),
    scratch_types=[
        pltpu.VMEM((NUM_LAYERS,), jnp.float32),
        pltpu.VMEM((NUM_LAYERS,), jnp.float32),
        pltpu.VMEM((NUM_LAYERS,), jnp.float32),
        pltpu.VMEM((NUM_LAYERS,), jnp.float32),
        pltpu.VMEM((ROWS_PER_W, SIZE), jnp.float32),
        pltpu.SemaphoreType.DMA,
    ],
)(_body)


@jax.jit
def kernel(params, ln_weight, ln_bias):
    masks, outputs = _sk(params, ln_weight, ln_bias)
    return masks, outputs


# speedup vs baseline: 1.0713x; 1.0370x over previous
import functools

import jax
import jax.numpy as jnp
from jax import lax
from jax.experimental import pallas as pl
from jax.experimental.pallas import tpu as pltpu
from jax.experimental.pallas import tpu_sc as plsc

NUM_LAYERS = 128
SIZE = 4096
SOFT = 204
INV_DENOM = 1.0 / (2.0 * SOFT)
OFFSET = 3.0
EPS = 1e-5
L = 16
NC, NS = 2, 16
NW = NC * NS
ROWS_PER_W = NUM_LAYERS // NW
NVEC = SIZE // L
WINDOW = 40


def _vrsqrt(v):
    i = plsc.bitcast(v, jnp.int32)
    i = jnp.int32(0x5F3759DF) - lax.shift_right_arithmetic(i, 1)
    y = plsc.bitcast(i, jnp.float32)
    for _ in range(4):
        y = y * (1.5 - 0.5 * v * y * y)
    return y


def _body(p_hbm, w_hbm, b_hbm, masks_hbm, outs_hbm, p_v, w_v, b_v, o_v,
          row_v, sem):
    wid = lax.axis_index("s") * NC + lax.axis_index("c")

    cp_p = pltpu.async_copy(p_hbm.at[0], p_v, sem)
    cp_w = pltpu.async_copy(w_hbm, w_v, sem)
    cp_b = pltpu.async_copy(b_hbm, b_v, sem)
    cp_p.wait()

    acc = jnp.zeros((L,), jnp.float32)
    acc2 = jnp.zeros((L,), jnp.float32)
    xs = []
    for i in range(NUM_LAYERS // L):
        x = p_v[pl.ds(i * L, L)]
        xs.append(x)
        acc = acc + x
        acc2 = acc2 + x * x
    s = jnp.sum(acc)
    s2 = jnp.sum(acc2)
    mu = s * (1.0 / NUM_LAYERS)
    var = s2 * (1.0 / NUM_LAYERS) - mu * mu
    rstd = _vrsqrt(jnp.full((L,), var + EPS, jnp.float32))
    cp_w.wait()
    cp_b.wait()
    for i in range(NUM_LAYERS // L):
        xhat = (xs[i] - mu) * rstd
        y = xhat * w_v[pl.ds(i * L, L)] + b_v[pl.ds(i * L, L)] + OFFSET
        o_v[pl.ds(i * L, L)] = 1.0 / (1.0 + jnp.exp(-y))

    out_cp = None
    if True:
        @pl.when(wid == 0)
        def _():
            pltpu.async_copy(o_v, outs_hbm, sem)

    lane_scaled = (lax.convert_element_type(lax.iota(jnp.int32, L),
                                            jnp.float32) * INV_DENOM)
    ones_v = jnp.full((L,), 1.0, jnp.float32)
    zeros_v = jnp.zeros((L,), jnp.float32)
    copies = []
    for r in range(ROWS_PER_W):
        row = wid * ROWS_PER_W + r
        k_vec = plsc.load_gather(
            o_v, [jnp.full((L,), row, jnp.int32)]) * float(SIZE)
        a_vec = k_vec * INV_DENOM + 0.5 - lane_scaled

        @plsc.parallel_loop(jnp.int32(0), jnp.int32(NVEC), 8)
        def _rowloop(i, r=r, a_vec=a_vec):
            va = a_vec - lax.convert_element_type(i, jnp.float32) * (L * INV_DENOM)
            for v in range(8):
                val = jnp.clip(va - float(v * L * INV_DENOM), 0.0, 1.0)
                row_v[r, pl.ds((i + v) * L, L)] = val

        copies.append(pltpu.async_copy(row_v.at[r], masks_hbm.at[row], sem))

    for c in copies:
        c.wait()

    @pl.when(wid == 0)
    def _():
        pltpu.make_async_copy(o_v, outs_hbm, sem).wait()


_sk = functools.partial(
    pl.kernel,
    out_type=(
        jax.ShapeDtypeStruct((NUM_LAYERS, SIZE), jnp.float32),
        jax.ShapeDtypeStruct((NUM_LAYERS,), jnp.float32),
    ),
    mesh=plsc.VectorSubcoreMesh(core_axis_name="c", subcore_axis_name="s",
                                num_cores=NC, num_subcores=NS),
    compiler_params=pltpu.CompilerParams(
        needs_layout_passes=False,
        skip_device_barrier=True,
        disable_bounds_checks=True,
        disable_semaphore_checks=True,
    ),
    scratch_types=[
        pltpu.VMEM((NUM_LAYERS,), jnp.float32),
        pltpu.VMEM((NUM_LAYERS,), jnp.float32),
        pltpu.VMEM((NUM_LAYERS,), jnp.float32),
        pltpu.VMEM((NUM_LAYERS,), jnp.float32),
        pltpu.VMEM((ROWS_PER_W, SIZE), jnp.float32),
        pltpu.SemaphoreType.DMA,
    ],
)(_body)


@jax.jit
def kernel(params, ln_weight, ln_bias):
    masks, outputs = _sk(params, ln_weight, ln_bias)
    return masks, outputs
